# baseline (speedup 1.0000x reference)
import functools

import jax
import jax.numpy as jnp
from jax import lax
from jax.experimental import pallas as pl
from jax.experimental.pallas import tpu as pltpu
from jax.experimental.pallas import tpu_sc as plsc

NUM_CORES = 2
NUM_SUBCORES = 16
NW = NUM_CORES * NUM_SUBCORES
LANES = 16


def _sc_gather(tt, idx_t):
    fields, dim, vocab = tt.shape
    batch = idx_t.shape[1]
    rows = fields * dim
    rows_per_w = rows // NW
    mesh = plsc.VectorSubcoreMesh(core_axis_name="c", subcore_axis_name="s")

    half = 50048
    rest = vocab - half

    @functools.partial(
        pl.kernel,
        mesh=mesh,
        compiler_params=pltpu.CompilerParams(
            use_tc_tiling_on_sc=True, needs_layout_passes=False
        ),
        out_type=jax.ShapeDtypeStruct((fields, dim, batch), jnp.float32),
        scratch_types=[
            pltpu.VMEM((half,), jnp.float32),
            pltpu.VMEM((rest,), jnp.float32),
            pltpu.VMEM((batch,), jnp.int32),
            pltpu.VMEM((batch,), jnp.float32),
            pltpu.SemaphoreType.DMA,
            pltpu.SemaphoreType.DMA,
        ],
    )
    def k(tt_hbm, idx_hbm, out_hbm, buf_a, buf_b, idxbuf, outbuf, sem_a, sem_b):
        wid = lax.axis_index("s") * NUM_CORES + lax.axis_index("c")
        row0 = wid * rows_per_w

        UNROLL = 8

        def dma_a(r):
            f = r // dim
            d = r % dim
            return pltpu.make_async_copy(
                tt_hbm.at[f, d, pl.ds(0, half)], buf_a, sem_a
            )

        def dma_b(r):
            f = r // dim
            d = r % dim
            return pltpu.make_async_copy(
                tt_hbm.at[f, d, pl.ds(half, rest)], buf_b, sem_b
            )

        dma_a(row0).start()
        dma_b(row0).start()

        def per_row(j, carry):
            r = row0 + j
            f = r // dim
            d = r % dim

            @pl.when((j == 0) | (d == 0))
            def _():
                pltpu.sync_copy(idx_hbm.at[f], idxbuf)

            dma_a(r).wait()

            def gather_lo(i, c):
                for u in range(UNROLL):
                    o = (i * UNROLL + u) * LANES
                    idxv = idxbuf[pl.ds(o, LANES)]
                    outbuf[pl.ds(o, LANES)] = plsc.load_gather(buf_a, [idxv])
                return c

            lax.fori_loop(0, batch // (LANES * UNROLL), gather_lo, 0)

            @pl.when(j + 1 < rows_per_w)
            def _():
                dma_a(r + 1).start()

            dma_b(r).wait()

            def gather_hi(i, c):
                for u in range(UNROLL):
                    o = (i * UNROLL + u) * LANES
                    idxv = idxbuf[pl.ds(o, LANES)]
                    hi = plsc.load_gather(
                        buf_b, [jnp.maximum(idxv - half, 0)]
                    )
                    lo = outbuf[pl.ds(o, LANES)]
                    outbuf[pl.ds(o, LANES)] = jnp.where(idxv >= half, hi, lo)
                return c

            lax.fori_loop(0, batch // (LANES * UNROLL), gather_hi, 0)

            @pl.when(j + 1 < rows_per_w)
            def _():
                dma_b(r + 1).start()

            pltpu.sync_copy(outbuf, out_hbm.at[f, d])
            return carry

        lax.fori_loop(0, rows_per_w, per_row, 0)

    return k(tt, idx_t)


def kernel(inputs, tables):
    fields, vocab, dim = tables.shape
    batch = inputs.shape[0]
    tt = tables.transpose(0, 2, 1)
    idx_t = inputs.T
    out = _sc_gather(tt, idx_t)
    return out.transpose(2, 0, 1)


# speedup vs baseline: 1.0466x; 1.0466x over previous
import functools

import jax
import jax.numpy as jnp
from jax import lax
from jax.experimental import pallas as pl
from jax.experimental.pallas import tpu as pltpu
from jax.experimental.pallas import tpu_sc as plsc

NUM_CORES = 2
NUM_SUBCORES = 16
NW = NUM_CORES * NUM_SUBCORES
LANES = 16


def _sc_gather(tt, idx_t):
    fields, dim, vocab = tt.shape
    batch = idx_t.shape[1]
    rows = fields * dim
    rows_per_w = rows // NW
    mesh = plsc.VectorSubcoreMesh(core_axis_name="c", subcore_axis_name="s")

    half = 50048
    rest = vocab - half

    @functools.partial(
        pl.kernel,
        mesh=mesh,
        compiler_params=pltpu.CompilerParams(
            use_tc_tiling_on_sc=True, needs_layout_passes=False
        ),
        out_type=jax.ShapeDtypeStruct((fields, dim, batch), jnp.float32),
        scratch_types=[
            pltpu.VMEM((half,), jnp.float32),
            pltpu.VMEM((rest,), jnp.float32),
            pltpu.VMEM((batch,), jnp.int32),
            pltpu.VMEM((batch,), jnp.float32),
            pltpu.SemaphoreType.DMA,
            pltpu.SemaphoreType.DMA,
        ],
    )
    def k(tt_hbm, idx_hbm, out_hbm, buf_a, buf_b, idxbuf, outbuf, sem_a, sem_b):
        wid = lax.axis_index("s") * NUM_CORES + lax.axis_index("c")
        row0 = wid * rows_per_w

        UNROLL = 8

        def dma_a(r):
            f = r // dim
            d = r % dim
            return pltpu.make_async_copy(
                tt_hbm.at[f, d, pl.ds(0, half)], buf_a, sem_a
            )

        def dma_b(r):
            f = r // dim
            d = r % dim
            return pltpu.make_async_copy(
                tt_hbm.at[f, d, pl.ds(half, rest)], buf_b, sem_b
            )

        dma_a(row0).start()
        dma_b(row0).start()

        def per_row(j, carry):
            r = row0 + j
            f = r // dim
            d = r % dim

            @pl.when((j == 0) | (d == 0))
            def _():
                pltpu.sync_copy(idx_hbm.at[f], idxbuf)

            dma_a(r).wait()

            def gather_lo(i, c):
                for u in range(UNROLL):
                    o = (i * UNROLL + u) * LANES
                    idxv = idxbuf[pl.ds(o, LANES)]
                    outbuf[pl.ds(o, LANES)] = plsc.load_gather(
                        buf_a, [jnp.minimum(idxv, half - 1)]
                    )
                return c

            lax.fori_loop(0, batch // (LANES * UNROLL), gather_lo, 0)

            @pl.when(j + 1 < rows_per_w)
            def _():
                dma_a(r + 1).start()

            dma_b(r).wait()

            def gather_hi(i, c):
                for u in range(UNROLL):
                    o = (i * UNROLL + u) * LANES
                    idxv = idxbuf[pl.ds(o, LANES)]
                    hi = plsc.load_gather(
                        buf_b,
                        [jnp.clip(idxv - half, 0, rest - 1)],
                    )
                    lo = outbuf[pl.ds(o, LANES)]
                    outbuf[pl.ds(o, LANES)] = jnp.where(idxv >= half, hi, lo)
                return c

            lax.fori_loop(0, batch // (LANES * UNROLL), gather_hi, 0)

            @pl.when(j + 1 < rows_per_w)
            def _():
                dma_b(r + 1).start()

            pltpu.sync_copy(outbuf, out_hbm.at[f, d])
            return carry

        lax.fori_loop(0, rows_per_w, per_row, 0)

    return k(tt, idx_t)


def kernel(inputs, tables):
    fields, vocab, dim = tables.shape
    batch = inputs.shape[0]
    tt = tables.transpose(0, 2, 1)
    idx_t = inputs.T
    out = _sc_gather(tt, idx_t)
    return out.transpose(2, 0, 1)


# speedup vs baseline: 1.0471x; 1.0005x over previous
import functools

import jax
import jax.numpy as jnp
from jax import lax
from jax.experimental import pallas as pl
from jax.experimental.pallas import tpu as pltpu
from jax.experimental.pallas import tpu_sc as plsc

NUM_CORES = 2
NUM_SUBCORES = 16
NW = NUM_CORES * NUM_SUBCORES
LANES = 16


def _sc_gather(tt, idx_t):
    fields, dim, vocab = tt.shape
    batch = idx_t.shape[1]
    rows = fields * dim
    rows_per_w = rows // NW
    mesh = plsc.VectorSubcoreMesh(core_axis_name="c", subcore_axis_name="s")

    half = 50048
    rest = vocab - half

    @functools.partial(
        pl.kernel,
        mesh=mesh,
        compiler_params=pltpu.CompilerParams(
            use_tc_tiling_on_sc=True, needs_layout_passes=False
        ),
        out_type=jax.ShapeDtypeStruct((fields, dim, batch), jnp.float32),
        scratch_types=[
            pltpu.VMEM((half,), jnp.float32),
            pltpu.VMEM((rest,), jnp.float32),
            pltpu.VMEM((batch,), jnp.int32),
            pltpu.VMEM((batch,), jnp.float32),
            pltpu.SemaphoreType.DMA,
            pltpu.SemaphoreType.DMA,
        ],
    )
    def k(tt_hbm, idx_hbm, out_hbm, buf_a, buf_b, idxbuf, outbuf, sem_a, sem_b):
        wid = lax.axis_index("s") * NUM_CORES + lax.axis_index("c")
        row0 = wid * rows_per_w

        UNROLL = 16

        def dma_a(r):
            f = r // dim
            d = r % dim
            return pltpu.make_async_copy(
                tt_hbm.at[f, d, pl.ds(0, half)], buf_a, sem_a
            )

        def dma_b(r):
            f = r // dim
            d = r % dim
            return pltpu.make_async_copy(
                tt_hbm.at[f, d, pl.ds(half, rest)], buf_b, sem_b
            )

        dma_a(row0).start()
        dma_b(row0).start()

        def per_row(j, carry):
            r = row0 + j
            f = r // dim
            d = r % dim

            @pl.when((j == 0) | (d == 0))
            def _():
                pltpu.sync_copy(idx_hbm.at[f], idxbuf)

            dma_a(r).wait()

            def gather_lo(i, c):
                for u in range(UNROLL):
                    o = (i * UNROLL + u) * LANES
                    idxv = idxbuf[pl.ds(o, LANES)]
                    outbuf[pl.ds(o, LANES)] = plsc.load_gather(
                        buf_a, [jnp.minimum(idxv, half - 1)]
                    )
                return c

            lax.fori_loop(0, batch // (LANES * UNROLL), gather_lo, 0)

            @pl.when(j + 1 < rows_per_w)
            def _():
                dma_a(r + 1).start()

            dma_b(r).wait()

            def gather_hi(i, c):
                for u in range(UNROLL):
                    o = (i * UNROLL + u) * LANES
                    idxv = idxbuf[pl.ds(o, LANES)]
                    hi = plsc.load_gather(
                        buf_b,
                        [jnp.clip(idxv - half, 0, rest - 1)],
                    )
                    lo = outbuf[pl.ds(o, LANES)]
                    outbuf[pl.ds(o, LANES)] = jnp.where(idxv >= half, hi, lo)
                return c

            lax.fori_loop(0, batch // (LANES * UNROLL), gather_hi, 0)

            @pl.when(j + 1 < rows_per_w)
            def _():
                dma_b(r + 1).start()

            pltpu.sync_copy(outbuf, out_hbm.at[f, d])
            return carry

        lax.fori_loop(0, rows_per_w, per_row, 0)

    return k(tt, idx_t)


def kernel(inputs, tables):
    fields, vocab, dim = tables.shape
    batch = inputs.shape[0]
    tt = tables.transpose(0, 2, 1)
    idx_t = inputs.T
    out = _sc_gather(tt, idx_t)
    return out.transpose(2, 0, 1)
